# baseline (speedup 1.0000x reference)
import functools

import jax
import jax.numpy as jnp
from jax import lax
from jax.experimental import pallas as pl
from jax.experimental.pallas import tpu as pltpu
from jax.experimental.pallas import tpu_sc as plsc

_B = 16384
_N = 1000
_NC = 2
_NS = 16
_CPW = _B // _NS
_RCHUNK = 40
_NGRP = _CPW // 16
_ROWS0 = 440
_CHUNKS0 = _ROWS0 // _RCHUNK
_CHUNKS1 = (_N - _ROWS0) // _RCHUNK


def _make_sc_one_hot():
    mesh = plsc.VectorSubcoreMesh(core_axis_name="c", subcore_axis_name="s")

    @functools.partial(
        pl.kernel,
        out_type=jax.ShapeDtypeStruct((_N, _B), jnp.float32),
        mesh=mesh,
        compiler_params=pltpu.CompilerParams(needs_layout_passes=False),
        scratch_types=[
            pltpu.VMEM((_CPW,), jnp.int32),
            pltpu.VMEM((_CPW,), jnp.float32),
            pltpu.VMEM((_RCHUNK, _CPW), jnp.float32),
            pltpu.VMEM((_RCHUNK, _CPW), jnp.float32),
            pltpu.SemaphoreType.DMA,
            pltpu.SemaphoreType.DMA,
        ],
    )
    def one_hot_kernel(labels_hbm, src_hbm, zeros_hbm, out_hbm,
                       lab_v, src_v, buf0, buf1, sem0, sem1):
        cid = lax.axis_index("c")
        col0 = lax.axis_index("s") * _CPW

        pltpu.sync_copy(labels_hbm.at[pl.ds(col0, _CPW)], lab_v)
        pltpu.sync_copy(src_hbm.at[pl.ds(col0, _CPW)], src_v)

        zeros16 = jnp.zeros((16,), jnp.float32)
        iota16 = lax.iota(jnp.int32, 16)
        bufs = (buf0, buf1)
        sems = (sem0, sem1)

        def scan_chunk(buf, new_r0, old_r0):
            def body(g, carry):
                lab16 = lab_v[pl.ds(g * 16, 16)]
                col16 = iota16 + g * 16
                if old_r0 is not None:
                    old_row = lab16 - old_r0
                    old_msk = (old_row >= 0) & (old_row < _RCHUNK)
                    plsc.store_scatter(buf, [old_row, col16], zeros16,
                                       mask=old_msk)
                new_row = lab16 - new_r0
                new_msk = (new_row >= 0) & (new_row < _RCHUNK)
                plsc.store_scatter(buf, [new_row, col16],
                                   src_v[pl.ds(g * 16, 16)], mask=new_msk)
                return carry

            lax.fori_loop(0, _NGRP, body, 0)

        def start_dma(buf, r0, sem):
            dst = out_hbm.at[pl.ds(r0, _RCHUNK), pl.ds(col0, _CPW)]
            pltpu.async_copy(buf, dst, sem)

        def wait_dma(buf, sem):
            pltpu.make_async_copy(
                buf, out_hbm.at[pl.ds(0, _RCHUNK), pl.ds(col0, _CPW)], sem
            ).wait()

        def pipeline(base, nchunk):
            def r0_of(c):
                return base + c * _RCHUNK

            for b in range(2):
                pltpu.sync_copy(zeros_hbm, bufs[b])
                scan_chunk(bufs[b], r0_of(b), None)
                start_dma(bufs[b], r0_of(b), sems[b])

            def round_body(g, carry):
                c0 = 2 + g * 2
                for b in range(2):
                    r0 = r0_of(c0 + b)
                    wait_dma(bufs[b], sems[b])
                    scan_chunk(bufs[b], r0, r0 - 2 * _RCHUNK)
                    start_dma(bufs[b], r0, sems[b])
                return carry

            lax.fori_loop(0, (nchunk - 2) // 2, round_body, 0)

            if nchunk % 2:
                r0 = r0_of(nchunk - 1)
                wait_dma(buf0, sem0)
                scan_chunk(buf0, r0, r0 - 2 * _RCHUNK)
                start_dma(buf0, r0, sem0)
            wait_dma(buf1, sem1)
            wait_dma(buf0, sem0)

        @pl.when(cid == 0)
        def _():
            pipeline(0, _CHUNKS0)

        @pl.when(cid == 1)
        def _():
            pipeline(_ROWS0, _CHUNKS1)

    return one_hot_kernel


_sc_one_hot = _make_sc_one_hot()


def kernel(labels, src_ones):
    labels_flat = labels.reshape(_B).astype(jnp.int32)
    src_flat = src_ones.reshape(_B).astype(jnp.float32)
    zeros_chunk = jnp.zeros((_RCHUNK, _CPW), jnp.float32)
    out_t = _sc_one_hot(labels_flat, src_flat, zeros_chunk)
    return out_t.T


# speedup vs baseline: 1.1078x; 1.1078x over previous
import functools

import jax
import jax.numpy as jnp
from jax import lax
from jax.experimental import pallas as pl
from jax.experimental.pallas import tpu as pltpu
from jax.experimental.pallas import tpu_sc as plsc

_B = 16384
_N = 1000
_NC = 2
_NS = 16
_CPW = _B // _NS
_RCHUNK = 40
_NGRP = _CPW // 16
_ROWS0 = 440
_CHUNKS0 = _ROWS0 // _RCHUNK
_CHUNKS1 = (_N - _ROWS0) // _RCHUNK


def _make_sc_one_hot():
    mesh = plsc.VectorSubcoreMesh(core_axis_name="c", subcore_axis_name="s")

    @functools.partial(
        pl.kernel,
        out_type=jax.ShapeDtypeStruct((_N, _B), jnp.float32),
        mesh=mesh,
        compiler_params=pltpu.CompilerParams(needs_layout_passes=False),
        scratch_types=[
            pltpu.VMEM((_CPW,), jnp.int32),
            pltpu.VMEM((_CPW,), jnp.float32),
            pltpu.VMEM((_RCHUNK, _CPW), jnp.float32),
            pltpu.VMEM((_RCHUNK, _CPW), jnp.float32),
            pltpu.SemaphoreType.DMA,
            pltpu.SemaphoreType.DMA,
        ],
    )
    def one_hot_kernel(labels_hbm, src_hbm, zeros_hbm, out_hbm,
                       lab_v, src_v, buf0, buf1, sem0, sem1):
        cid = lax.axis_index("c")
        col0 = lax.axis_index("s") * _CPW

        pltpu.sync_copy(labels_hbm.at[pl.ds(col0, _CPW)], lab_v)
        pltpu.sync_copy(src_hbm.at[pl.ds(col0, _CPW)], src_v)
        pltpu.sync_copy(zeros_hbm, buf0)
        pltpu.sync_copy(zeros_hbm, buf1)

        zeros16 = jnp.zeros((16,), jnp.float32)
        iota16 = lax.iota(jnp.int32, 16)
        bufs = (buf0, buf1)
        sems = (sem0, sem1)

        def scan_chunk(buf, new_r0, old_r0):
            def body(g, carry):
                lab16 = lab_v[pl.ds(g * 16, 16)]
                col16 = iota16 + g * 16
                if old_r0 is not None:
                    old_row = lab16 - old_r0
                    old_msk = (old_row >= 0) & (old_row < _RCHUNK)
                    plsc.store_scatter(buf, [old_row, col16], zeros16,
                                       mask=old_msk)
                new_row = lab16 - new_r0
                new_msk = (new_row >= 0) & (new_row < _RCHUNK)
                plsc.store_scatter(buf, [new_row, col16],
                                   src_v[pl.ds(g * 16, 16)], mask=new_msk)
                return carry

            lax.fori_loop(0, _NGRP, body, 0)

        def start_dma(buf, r0, sem):
            dst = out_hbm.at[pl.ds(r0, _RCHUNK), pl.ds(col0, _CPW)]
            pltpu.async_copy(buf, dst, sem)

        def wait_dma(buf, sem):
            pltpu.make_async_copy(
                buf, out_hbm.at[pl.ds(0, _RCHUNK), pl.ds(col0, _CPW)], sem
            ).wait()

        def pipeline(base, nchunk):
            def r0_of(c):
                return base + c * _RCHUNK

            for b in range(2):
                scan_chunk(bufs[b], r0_of(b), None)
                start_dma(bufs[b], r0_of(b), sems[b])

            def round_body(g, carry):
                c0 = 2 + g * 2
                for b in range(2):
                    r0 = r0_of(c0 + b)
                    wait_dma(bufs[b], sems[b])
                    scan_chunk(bufs[b], r0, r0 - 2 * _RCHUNK)
                    start_dma(bufs[b], r0, sems[b])
                return carry

            lax.fori_loop(0, (nchunk - 2) // 2, round_body, 0)

            if nchunk % 2:
                r0 = r0_of(nchunk - 1)
                wait_dma(buf0, sem0)
                scan_chunk(buf0, r0, r0 - 2 * _RCHUNK)
                start_dma(buf0, r0, sem0)
            wait_dma(buf1, sem1)
            wait_dma(buf0, sem0)

        @pl.when(cid == 0)
        def _():
            pipeline(0, _CHUNKS0)

        @pl.when(cid == 1)
        def _():
            pipeline(_ROWS0, _CHUNKS1)

    return one_hot_kernel


_sc_one_hot = _make_sc_one_hot()


def kernel(labels, src_ones):
    labels_flat = labels.reshape(_B).astype(jnp.int32)
    src_flat = src_ones.reshape(_B).astype(jnp.float32)
    zeros_chunk = jnp.zeros((_RCHUNK, _CPW), jnp.float32)
    out_t = _sc_one_hot(labels_flat, src_flat, zeros_chunk)
    return out_t.T


# speedup vs baseline: 1.1102x; 1.0021x over previous
import functools

import jax
import jax.numpy as jnp
from jax import lax
from jax.experimental import pallas as pl
from jax.experimental.pallas import tpu as pltpu
from jax.experimental.pallas import tpu_sc as plsc

_B = 16384
_N = 1000
_NC = 2
_NS = 16
_CPW = _B // _NS
_RCHUNK = 40
_NGRP = _CPW // 16
_ROWS0 = 440
_CHUNKS0 = _ROWS0 // _RCHUNK
_CHUNKS1 = (_N - _ROWS0) // _RCHUNK


def _make_sc_one_hot():
    mesh = plsc.VectorSubcoreMesh(core_axis_name="c", subcore_axis_name="s")

    @functools.partial(
        pl.kernel,
        out_type=jax.ShapeDtypeStruct((_N, _B), jnp.float32),
        mesh=mesh,
        compiler_params=pltpu.CompilerParams(needs_layout_passes=False),
        scratch_types=[
            pltpu.VMEM((_CPW,), jnp.int32),
            pltpu.VMEM((_CPW,), jnp.float32),
            pltpu.VMEM((_RCHUNK, _CPW), jnp.float32),
            pltpu.VMEM((_RCHUNK, _CPW), jnp.float32),
            pltpu.SemaphoreType.DMA,
            pltpu.SemaphoreType.DMA,
        ],
    )
    def one_hot_kernel(labels_hbm, src_hbm, zeros_hbm, out_hbm,
                       lab_v, src_v, buf0, buf1, sem0, sem1):
        cid = lax.axis_index("c")
        col0 = lax.axis_index("s") * _CPW

        pltpu.sync_copy(labels_hbm.at[pl.ds(col0, _CPW)], lab_v)
        pltpu.sync_copy(src_hbm.at[pl.ds(col0, _CPW)], src_v)
        pltpu.sync_copy(zeros_hbm, buf0)
        pltpu.sync_copy(zeros_hbm, buf1)

        zeros16 = jnp.zeros((16,), jnp.float32)
        iota16 = lax.iota(jnp.int32, 16)
        bufs = (buf0, buf1)
        sems = (sem0, sem1)

        def scan_chunk(buf, new_r0, old_r0):
            def body(g, carry):
                lab16 = lab_v[pl.ds(g * 16, 16)]
                col16 = iota16 + g * 16
                if old_r0 is not None:
                    old_row = lab16 - old_r0
                    old_msk = (old_row >= 0) & (old_row < _RCHUNK)
                    plsc.store_scatter(buf, [old_row, col16], zeros16,
                                       mask=old_msk)
                new_row = lab16 - new_r0
                new_msk = (new_row >= 0) & (new_row < _RCHUNK)
                plsc.store_scatter(buf, [new_row, col16],
                                   src_v[pl.ds(g * 16, 16)], mask=new_msk)
                return carry

            lax.fori_loop(0, _NGRP, body, 0)

        def start_dma(buf, r0, sem):
            dst = out_hbm.at[pl.ds(r0, _RCHUNK), pl.ds(col0, _CPW)]
            pltpu.async_copy(buf, dst, sem)

        def wait_dma(buf, sem):
            pltpu.make_async_copy(
                buf, out_hbm.at[pl.ds(0, _RCHUNK), pl.ds(col0, _CPW)], sem
            ).wait()

        base = jnp.where(cid == 0, 0, _ROWS0)
        nchunk = jnp.where(cid == 0, _CHUNKS0, _CHUNKS1)

        def r0_of(c):
            return base + c * _RCHUNK

        for b in range(2):
            scan_chunk(bufs[b], r0_of(b), None)
            start_dma(bufs[b], r0_of(b), sems[b])

        def round_body(g, carry):
            c0 = 2 + g * 2
            for b in range(2):
                r0 = r0_of(c0 + b)
                wait_dma(bufs[b], sems[b])
                scan_chunk(bufs[b], r0, r0 - 2 * _RCHUNK)
                start_dma(bufs[b], r0, sems[b])
            return carry

        lax.fori_loop(0, (nchunk - 2) // 2, round_body, 0)

        @pl.when(nchunk % 2 == 1)
        def _():
            r0 = r0_of(nchunk - 1)
            wait_dma(buf0, sem0)
            scan_chunk(buf0, r0, r0 - 2 * _RCHUNK)
            start_dma(buf0, r0, sem0)

        wait_dma(buf1, sem1)
        wait_dma(buf0, sem0)

    return one_hot_kernel


_sc_one_hot = _make_sc_one_hot()


def kernel(labels, src_ones):
    labels_flat = labels.reshape(_B).astype(jnp.int32)
    src_flat = src_ones.reshape(_B).astype(jnp.float32)
    zeros_chunk = jnp.zeros((_RCHUNK, _CPW), jnp.float32)
    out_t = _sc_one_hot(labels_flat, src_flat, zeros_chunk)
    return out_t.T
